# Initial kernel scaffold; baseline (speedup 1.0000x reference)
import jax
import jax.numpy as jnp
from jax.experimental import pallas as pl


def kernel(z, emb_weight, compress_w, compress_b, expand_w, expand_b):
    raise NotImplementedError("write your pallas kernel here")


# speedup vs baseline: 1.0927x; 1.0927x over previous
import jax
import jax.numpy as jnp
from jax import lax
from jax.experimental import pallas as pl
from jax.experimental.pallas import tpu as pltpu
from jax.experimental.pallas import tpu_sc as plsc

N_TOKENS = 8192
D_IN = 384
D_CODE = 32
N_CODES = 8192
TILE_T = 256
N_TILES = N_TOKENS // TILE_T
TILE_C = 1024
N_TILES_C = N_TOKENS // TILE_C

SC_CORES = 2
SC_SUBCORES = 16
SC_WORKERS = SC_CORES * SC_SUBCORES
ROWS_PER_WORKER = N_TOKENS // SC_WORKERS
GATHER_CHUNK = 128
N_CHUNKS = ROWS_PER_WORKER // GATHER_CHUNK


def _argmin_kernel(z_ref, cwt_ref, cb_ref, embt_ref, idx_ref):
    zc = jnp.dot(z_ref[...], cwt_ref[...],
                 preferred_element_type=jnp.float32) + cb_ref[...]
    embt = embt_ref[...]
    esq = jnp.sum(embt * embt, axis=0, keepdims=True)
    asq = jnp.sum(zc * zc, axis=1, keepdims=True)
    s = jnp.dot(zc, embt, preferred_element_type=jnp.float32)
    d = (asq + esq) - 2.0 * s
    dmin = jnp.min(d, axis=1, keepdims=True)
    iota = lax.broadcasted_iota(jnp.int32, d.shape, 1)
    idx = jnp.min(jnp.where(d == dmin, iota, N_CODES), axis=1)
    idx_ref[...] = idx.astype(jnp.int32).reshape(1, 1, TILE_T)


def _expand_kernel(zq_ref, z_ref, cwt_ref, cb_ref, ewt_ref, eb_ref,
                   out_ref, loss_ref):
    i = pl.program_id(0)
    zq = zq_ref[...]
    zc = jnp.dot(z_ref[...], cwt_ref[...],
                 preferred_element_type=jnp.float32) + cb_ref[...]
    diff = zq - zc
    part = (3.0 / (N_TOKENS * D_CODE)) * jnp.sum(diff * diff)

    @pl.when(i == 0)
    def _():
        loss_ref[...] = jnp.zeros_like(loss_ref)

    loss_ref[...] += part.reshape(1, 1)
    out_ref[...] = jnp.dot(zq, ewt_ref[...],
                           preferred_element_type=jnp.float32) + eb_ref[...]


def _sc_gather(emb_hbm, idx_hbm, zq_hbm, idx_v, rows_v, sem):
    wid = lax.axis_index("s") * SC_CORES + lax.axis_index("c")
    base = wid * ROWS_PER_WORKER
    for j in range(N_CHUNKS):
        pltpu.sync_copy(idx_hbm.at[pl.ds(base + j * GATHER_CHUNK,
                                         GATHER_CHUNK)], idx_v.at[j])
    for j in range(N_CHUNKS):
        pltpu.async_copy(emb_hbm.at[idx_v.at[j]],
                         rows_v.at[pl.ds(j * GATHER_CHUNK, GATHER_CHUNK)],
                         sem).wait()
    pltpu.sync_copy(rows_v, zq_hbm.at[pl.ds(base, ROWS_PER_WORKER)])


def kernel(z, emb_weight, compress_w, compress_b, expand_w, expand_b):
    z2d = z.reshape(N_TOKENS, D_IN)
    cwt = compress_w.T
    embt = emb_weight.T
    ewt = expand_w.T
    cb = compress_b.reshape(1, D_CODE)
    eb = expand_b.reshape(1, D_IN)

    idx3 = pl.pallas_call(
        _argmin_kernel,
        grid=(N_TILES,),
        in_specs=[
            pl.BlockSpec((TILE_T, D_IN), lambda i: (i, 0)),
            pl.BlockSpec((D_IN, D_CODE), lambda i: (0, 0)),
            pl.BlockSpec((1, D_CODE), lambda i: (0, 0)),
            pl.BlockSpec((D_CODE, N_CODES), lambda i: (0, 0)),
        ],
        out_specs=pl.BlockSpec((1, 1, TILE_T), lambda i: (i, 0, 0)),
        out_shape=jax.ShapeDtypeStruct((N_TILES, 1, TILE_T), jnp.int32),
    )(z2d, cwt, cb, embt)
    indices = idx3.reshape(N_TOKENS)

    gather = pl.kernel(
        _sc_gather,
        out_type=jax.ShapeDtypeStruct((N_TOKENS, D_CODE), jnp.float32),
        mesh=plsc.VectorSubcoreMesh(core_axis_name="c", subcore_axis_name="s"),
        scratch_types=[
            pltpu.VMEM((N_CHUNKS, GATHER_CHUNK), jnp.int32),
            pltpu.VMEM((ROWS_PER_WORKER, D_CODE), jnp.float32),
            pltpu.SemaphoreType.DMA,
        ],
        compiler_params=pltpu.CompilerParams(use_tc_tiling_on_sc=False),
    )
    z_q = gather(emb_weight, indices)

    out2d, loss11 = pl.pallas_call(
        _expand_kernel,
        grid=(N_TILES_C,),
        in_specs=[
            pl.BlockSpec((TILE_C, D_CODE), lambda i: (i, 0)),
            pl.BlockSpec((TILE_C, D_IN), lambda i: (i, 0)),
            pl.BlockSpec((D_IN, D_CODE), lambda i: (0, 0)),
            pl.BlockSpec((1, D_CODE), lambda i: (0, 0)),
            pl.BlockSpec((D_CODE, D_IN), lambda i: (0, 0)),
            pl.BlockSpec((1, D_IN), lambda i: (0, 0)),
        ],
        out_specs=[
            pl.BlockSpec((TILE_C, D_IN), lambda i: (i, 0)),
            pl.BlockSpec((1, 1), lambda i: (0, 0)),
        ],
        out_shape=[
            jax.ShapeDtypeStruct((N_TOKENS, D_IN), jnp.float32),
            jax.ShapeDtypeStruct((1, 1), jnp.float32),
        ],
    )(z_q, z2d, cwt, cb, ewt, eb)

    out = out2d.reshape(z.shape)
    loss = loss11.reshape(())
    return (out, loss)
